# baseline (speedup 1.0000x reference)
import jax
import jax.numpy as jnp
from jax.experimental import pallas as pl
from jax.experimental.pallas import tpu as pltpu

_N = 10000
_D = 128
_BR = 400
_NRB = _N // _BR
_LOG2E = 1.4426950408889634


def _gat_kernel(in_ref, w_ref, wf_ref, adja_ref, adjb_ref, b_ref, out_ref,
                xv_s, f1_s, f2t_s):
    i = pl.program_id(0)

    @pl.when(i == 0)
    def _():
        x = jnp.dot(in_ref[...], w_ref[...],
                    preferred_element_type=jnp.float32)
        ff = jnp.dot(x, wf_ref[...], preferred_element_type=jnp.float32)
        ff = jnp.where(ff > 0, ff, jnp.exp(ff) - 1.0) * jnp.float32(_LOG2E)
        f1_s[...] = ff[:, 0:2]
        f2t_s[...] = ff[:, 2:4].T
        xv = jnp.concatenate(
            [x, jnp.ones((_N, 1), jnp.float32),
             jnp.zeros((_N, 127), jnp.float32)], axis=1)
        xv_s[...] = xv.astype(jnp.bfloat16)

    xv = xv_s[...]
    hb = _BR // 2
    for half, adj_ref in ((0, adja_ref), (1, adjb_ref)):
        adj = adj_ref[...].astype(jnp.bfloat16)
        res = None
        for h in range(2):
            srow = f1_s[pl.ds(i * _BR + half * hb, hb),
                        h:h + 1].astype(jnp.bfloat16)
            scol = f2t_s[h:h + 1, :].astype(jnp.bfloat16)
            t = srow + scol
            t = jnp.maximum(t, jnp.bfloat16(0.2) * t)
            e = jnp.exp2(adj * t)
            acc = jnp.dot(e, xv, preferred_element_type=jnp.float32)
            v = acc[:, 0:_D] / acc[:, _D:_D + 1] + b_ref[h:h + 1, :]
            v = jnp.where(v > 0, v, jnp.exp(v) - 1.0)
            res = v if res is None else res + v
        out_ref[pl.ds(half * hb, hb), :] = res * 0.5


def kernel(inputs, adj_mat, W, w1, w2, b):
    wf = jnp.concatenate(
        [w1[0], w1[1], w2[0], w2[1], jnp.zeros((_D, 4), jnp.float32)], axis=1)
    out = pl.pallas_call(
        _gat_kernel,
        grid=(_NRB,),
        in_specs=[pl.BlockSpec((_N, _D), lambda i: (0, 0)),
                  pl.BlockSpec((_D, _D), lambda i: (0, 0)),
                  pl.BlockSpec((_D, 8), lambda i: (0, 0)),
                  pl.BlockSpec((_BR // 2, _N), lambda i: (2 * i, 0)),
                  pl.BlockSpec((_BR // 2, _N), lambda i: (2 * i + 1, 0)),
                  pl.BlockSpec((2, _D), lambda i: (0, 0))],
        out_specs=pl.BlockSpec((_BR, _D), lambda i: (i, 0)),
        out_shape=jax.ShapeDtypeStruct((_N, _D), jnp.float32),
        scratch_shapes=[pltpu.VMEM((_N, 256), jnp.bfloat16),
                        pltpu.VMEM((_N, 2), jnp.float32),
                        pltpu.VMEM((2, _N), jnp.float32)],
    )(inputs, W, wf, adj_mat, adj_mat, b)
    return out


# speedup vs baseline: 1.0366x; 1.0366x over previous
import jax
import jax.numpy as jnp
from jax.experimental import pallas as pl
from jax.experimental.pallas import tpu as pltpu

_N = 10000
_D = 128
_BR = 400
_NRB = _N // _BR
_LOG2E = 1.4426950408889634


def _gat_kernel(in_ref, w_ref, wf_ref, adj_ref, b_ref, out_ref,
                xv_s, f1_s, f2t_s):
    i = pl.program_id(0)

    @pl.when(i == 0)
    def _():
        x = jnp.dot(in_ref[...], w_ref[...],
                    preferred_element_type=jnp.float32)
        ff = jnp.dot(x, wf_ref[...], preferred_element_type=jnp.float32)
        ff = jnp.where(ff > 0, ff, jnp.exp(ff) - 1.0) * jnp.float32(_LOG2E)
        f1_s[...] = ff[:, 0:2]
        f2t_s[...] = ff[:, 2:4].T
        xv = jnp.concatenate(
            [x, jnp.ones((_N, 1), jnp.float32),
             jnp.zeros((_N, 127), jnp.float32)], axis=1)
        xv_s[...] = xv.astype(jnp.bfloat16)

    adj = adj_ref[...].astype(jnp.bfloat16)
    xv = xv_s[...]
    res = None
    for h in range(2):
        srow = f1_s[pl.ds(i * _BR, _BR), h:h + 1].astype(jnp.bfloat16)
        scol = f2t_s[h:h + 1, :].astype(jnp.bfloat16)
        t = srow + scol
        t = jnp.maximum(t, jnp.bfloat16(0.2) * t)
        e = jnp.exp2(adj * t)
        acc = jnp.dot(e, xv, preferred_element_type=jnp.float32)
        v = acc[:, 0:_D] / acc[:, _D:_D + 1] + b_ref[h:h + 1, :]
        v = jnp.where(v > 0, v, jnp.exp(v) - 1.0)
        res = v if res is None else res + v
    out_ref[...] = res * 0.5


def kernel(inputs, adj_mat, W, w1, w2, b):
    wf = jnp.concatenate(
        [w1[0], w1[1], w2[0], w2[1], jnp.zeros((_D, 4), jnp.float32)], axis=1)
    out = pl.pallas_call(
        _gat_kernel,
        grid=(_NRB,),
        in_specs=[pl.BlockSpec((_N, _D), lambda i: (0, 0)),
                  pl.BlockSpec((_D, _D), lambda i: (0, 0)),
                  pl.BlockSpec((_D, 8), lambda i: (0, 0)),
                  pl.BlockSpec((_BR, _N), lambda i: (i, 0)),
                  pl.BlockSpec((2, _D), lambda i: (0, 0))],
        out_specs=pl.BlockSpec((_BR, _D), lambda i: (i, 0)),
        out_shape=jax.ShapeDtypeStruct((_N, _D), jnp.float32),
        scratch_shapes=[pltpu.VMEM((_N, 256), jnp.bfloat16),
                        pltpu.VMEM((_N, 2), jnp.float32),
                        pltpu.VMEM((2, _N), jnp.float32)],
    )(inputs, W, wf, adj_mat, b)
    return out
